# baseline (speedup 1.0000x reference)
import jax
import jax.numpy as jnp
from jax.experimental import pallas as pl


_SEQ_BLOCK = 1024


def _bcast_kernel(pos_ref, out_ref):
    out_ref[...] = jnp.broadcast_to(pos_ref[...][None], out_ref.shape)


def kernel(x, pos_embedding):
    batch, seq_len, head_dim = x.shape
    n_blocks = seq_len // _SEQ_BLOCK
    return pl.pallas_call(
        _bcast_kernel,
        grid=(n_blocks,),
        in_specs=[pl.BlockSpec((_SEQ_BLOCK, head_dim), lambda s: (s, 0))],
        out_specs=pl.BlockSpec(
            (batch, _SEQ_BLOCK, head_dim), lambda s: (0, s, 0)
        ),
        out_shape=jax.ShapeDtypeStruct(
            (batch, seq_len, head_dim), pos_embedding.dtype
        ),
    )(pos_embedding)


# speedup vs baseline: 1.0906x; 1.0906x over previous
import jax
import jax.numpy as jnp
from jax.experimental import pallas as pl
from jax.experimental.pallas import tpu as pltpu


_N_CHUNKS = 4


def _make_dma_kernel(batch, seq_len, head_dim):
    ch = seq_len // _N_CHUNKS

    def _dma_kernel(pos_ref, out_ref, vmem, in_sems, out_sems):
        for i in range(_N_CHUNKS):
            pltpu.make_async_copy(
                pos_ref.at[pl.ds(i * ch, ch)],
                vmem.at[pl.ds(i * ch, ch)],
                in_sems.at[i],
            ).start()
        for i in range(_N_CHUNKS):
            pltpu.make_async_copy(
                pos_ref.at[pl.ds(i * ch, ch)],
                vmem.at[pl.ds(i * ch, ch)],
                in_sems.at[i],
            ).wait()
            for b in range(batch):
                pltpu.make_async_copy(
                    vmem.at[pl.ds(i * ch, ch)],
                    out_ref.at[b, pl.ds(i * ch, ch)],
                    out_sems.at[b],
                ).start()
        for i in range(_N_CHUNKS):
            for b in range(batch):
                pltpu.make_async_copy(
                    vmem.at[pl.ds(i * ch, ch)],
                    out_ref.at[b, pl.ds(i * ch, ch)],
                    out_sems.at[b],
                ).wait()

    return _dma_kernel


def kernel(x, pos_embedding):
    batch, seq_len, head_dim = x.shape
    return pl.pallas_call(
        _make_dma_kernel(batch, seq_len, head_dim),
        in_specs=[pl.BlockSpec(memory_space=pl.ANY)],
        out_specs=pl.BlockSpec(memory_space=pl.ANY),
        out_shape=jax.ShapeDtypeStruct(
            (batch, seq_len, head_dim), pos_embedding.dtype
        ),
        scratch_shapes=[
            pltpu.VMEM((seq_len, head_dim), pos_embedding.dtype),
            pltpu.SemaphoreType.DMA((_N_CHUNKS,)),
            pltpu.SemaphoreType.DMA((batch,)),
        ],
    )(pos_embedding)


# speedup vs baseline: 1.1200x; 1.0269x over previous
import jax
import jax.numpy as jnp
from jax.experimental import pallas as pl
from jax.experimental.pallas import tpu as pltpu


_N_CHUNKS = 8


def _make_dma_kernel(batch, seq_len, head_dim):
    ch = seq_len // _N_CHUNKS

    def _dma_kernel(pos_ref, out_ref, vmem, in_sems, out_sems):
        for i in range(_N_CHUNKS):
            pltpu.make_async_copy(
                pos_ref.at[pl.ds(i * ch, ch)],
                vmem.at[pl.ds(i * ch, ch)],
                in_sems.at[i],
            ).start()
        for i in range(_N_CHUNKS):
            pltpu.make_async_copy(
                pos_ref.at[pl.ds(i * ch, ch)],
                vmem.at[pl.ds(i * ch, ch)],
                in_sems.at[i],
            ).wait()
            for b in range(batch):
                pltpu.make_async_copy(
                    vmem.at[pl.ds(i * ch, ch)],
                    out_ref.at[b, pl.ds(i * ch, ch)],
                    out_sems.at[b],
                ).start()
        for i in range(_N_CHUNKS):
            for b in range(batch):
                pltpu.make_async_copy(
                    vmem.at[pl.ds(i * ch, ch)],
                    out_ref.at[b, pl.ds(i * ch, ch)],
                    out_sems.at[b],
                ).wait()

    return _dma_kernel


def kernel(x, pos_embedding):
    batch, seq_len, head_dim = x.shape
    return pl.pallas_call(
        _make_dma_kernel(batch, seq_len, head_dim),
        in_specs=[pl.BlockSpec(memory_space=pl.ANY)],
        out_specs=pl.BlockSpec(memory_space=pl.ANY),
        out_shape=jax.ShapeDtypeStruct(
            (batch, seq_len, head_dim), pos_embedding.dtype
        ),
        scratch_shapes=[
            pltpu.VMEM((seq_len, head_dim), pos_embedding.dtype),
            pltpu.SemaphoreType.DMA((_N_CHUNKS,)),
            pltpu.SemaphoreType.DMA((batch,)),
        ],
    )(pos_embedding)


# speedup vs baseline: 1.1299x; 1.0089x over previous
import jax
import jax.numpy as jnp
from jax.experimental import pallas as pl
from jax.experimental.pallas import tpu as pltpu


_N_CHUNKS = 16


def _make_dma_kernel(batch, seq_len, head_dim):
    ch = seq_len // _N_CHUNKS

    def _dma_kernel(pos_ref, out_ref, vmem, in_sems, out_sems):
        for i in range(_N_CHUNKS):
            pltpu.make_async_copy(
                pos_ref.at[pl.ds(i * ch, ch)],
                vmem.at[pl.ds(i * ch, ch)],
                in_sems.at[i],
            ).start()
        for i in range(_N_CHUNKS):
            pltpu.make_async_copy(
                pos_ref.at[pl.ds(i * ch, ch)],
                vmem.at[pl.ds(i * ch, ch)],
                in_sems.at[i],
            ).wait()
            for b in range(batch):
                pltpu.make_async_copy(
                    vmem.at[pl.ds(i * ch, ch)],
                    out_ref.at[b, pl.ds(i * ch, ch)],
                    out_sems.at[b],
                ).start()
        for i in range(_N_CHUNKS):
            for b in range(batch):
                pltpu.make_async_copy(
                    vmem.at[pl.ds(i * ch, ch)],
                    out_ref.at[b, pl.ds(i * ch, ch)],
                    out_sems.at[b],
                ).wait()

    return _dma_kernel


def kernel(x, pos_embedding):
    batch, seq_len, head_dim = x.shape
    return pl.pallas_call(
        _make_dma_kernel(batch, seq_len, head_dim),
        in_specs=[pl.BlockSpec(memory_space=pl.ANY)],
        out_specs=pl.BlockSpec(memory_space=pl.ANY),
        out_shape=jax.ShapeDtypeStruct(
            (batch, seq_len, head_dim), pos_embedding.dtype
        ),
        scratch_shapes=[
            pltpu.VMEM((seq_len, head_dim), pos_embedding.dtype),
            pltpu.SemaphoreType.DMA((_N_CHUNKS,)),
            pltpu.SemaphoreType.DMA((batch,)),
        ],
    )(pos_embedding)
